# Initial kernel scaffold; baseline (speedup 1.0000x reference)
import jax
import jax.numpy as jnp
from jax.experimental import pallas as pl


def kernel(inputs, codebook):
    raise NotImplementedError("write your pallas kernel here")


# speedup vs baseline: 1.4802x; 1.4802x over previous
import jax
import jax.numpy as jnp
from jax.experimental import pallas as pl
from jax.experimental.pallas import tpu as pltpu

EMBED = 64
K = 1024
BETA = 0.25
BLOCK = 512


def _vq_body(x_ref, c_ref, q_ref, codes_ref, loss_ref):
    i = pl.program_id(0)
    x = x_ref[...]
    c = c_ref[...]
    c2 = jnp.sum(c * c, axis=1)
    x2 = jnp.sum(x * x, axis=1, keepdims=True)
    xc = jax.lax.dot_general(x, c, (((1,), (1,)), ((), ())),
                             preferred_element_type=jnp.float32)
    scores = (x2 - 2.0 * xc) + c2[None, :]
    minv = jnp.min(scores, axis=1, keepdims=True)
    iota = jax.lax.broadcasted_iota(jnp.int32, scores.shape, 1)
    codes = jnp.min(jnp.where(scores == minv, iota, K), axis=1)
    onehot = (iota == codes[:, None]).astype(jnp.float32)
    q = jax.lax.dot_general(onehot, c, (((1,), (0,)), ((), ())),
                            preferred_element_type=jnp.float32)
    q_ref[...] = q
    codes_ref[...] = codes

    partial = jnp.sum(minv)

    @pl.when(i == 0)
    def _init():
        loss_ref[0, 0] = 0.0

    loss_ref[0, 0] += partial


def kernel(inputs, codebook):
    orig_shape = inputs.shape
    flat = inputs.reshape(-1, EMBED)
    n = flat.shape[0]
    q, codes, loss_acc = pl.pallas_call(
        _vq_body,
        grid=(n // BLOCK,),
        in_specs=[
            pl.BlockSpec((BLOCK, EMBED), lambda i: (i, 0)),
            pl.BlockSpec((K, EMBED), lambda i: (0, 0)),
        ],
        out_specs=[
            pl.BlockSpec((BLOCK, EMBED), lambda i: (i, 0)),
            pl.BlockSpec((BLOCK,), lambda i: (i,)),
            pl.BlockSpec(block_shape=(1, 1), index_map=lambda i: (0, 0),
                         memory_space=pltpu.SMEM),
        ],
        out_shape=[
            jax.ShapeDtypeStruct((n, EMBED), jnp.float32),
            jax.ShapeDtypeStruct((n,), jnp.int32),
            jax.ShapeDtypeStruct((1, 1), jnp.float32),
        ],
    )(flat, codebook)
    loss = loss_acc[0, 0] * (1.0 + BETA) / (n * EMBED)
    return q.reshape(orig_shape), codes.reshape(orig_shape[:-1]), loss
